# Initial kernel scaffold; baseline (speedup 1.0000x reference)
import jax
import jax.numpy as jnp
from jax.experimental import pallas as pl


def kernel(r, bond_indices, edge_numbers, hopping_params, overlap_params, atomic_radius_list):
    raise NotImplementedError("write your pallas kernel here")


# speedup vs baseline: 17.0813x; 17.0813x over previous
import jax
import jax.numpy as jnp
from jax import lax
from jax.experimental import pallas as pl

_BLK = 256
_NING = 13


def _body(r_ref, bt_ref, e0_ref, e1_ref, wt_ref, rad_ref, eye_ref, hop_ref, ov_ref):
    f32 = jnp.float32
    rrow = r_ref[0]
    bt = bt_ref[0]
    e0 = e0_ref[0]
    e1 = e1_ref[0]
    nbt = wt_ref.shape[1]
    blk = rrow.shape[1]

    iota_t = lax.broadcasted_iota(jnp.int32, (nbt, blk), 0)
    oht = jnp.where(iota_t == bt, 1.0, 0.0).astype(f32)
    iota_e = lax.broadcasted_iota(jnp.int32, (128, blk), 0)
    ohe = jnp.where(iota_e == e0, 1.0, 0.0) + jnp.where(iota_e == e1, 1.0, 0.0)

    g = lax.dot_general(wt_ref[...], oht, (((1,), (0,)), ((), ())),
                        preferred_element_type=f32)
    r0 = lax.dot_general(rad_ref[...], ohe, (((1,), (0,)), ((), ())),
                         preferred_element_type=f32)

    x = rrow / r0 - 1.0
    x2 = x * x
    lnr = jnp.log(r0 / rrow)
    fcut = 1.0 / (1.0 + jnp.exp((rrow - 5.0) * 5.0))

    eye = eye_ref[...]
    for t, out_ref in ((0, hop_ref), (1, ov_ref)):
        base = 64 * t
        g0 = g[base:base + 16]
        g1 = g[base + 16:base + 32]
        g2 = g[base + 32:base + 48]
        g3 = g[base + 48:base + 64]
        poly = g0 + g1 * x + g2 * x2
        a3 = 1.0 + jnp.abs(g3)
        out_t = poly * jnp.exp(a3 * lnr) * fcut
        out = lax.dot_general(eye, out_t, (((1,), (1,)), ((), ())),
                              preferred_element_type=f32)
        out_ref[...] = out[:, :_NING]


def kernel(r, bond_indices, edge_numbers, hopping_params, overlap_params,
           atomic_radius_list):
    f32 = jnp.float32
    B = r.shape[0]
    blk = _BLK
    nblk = B // blk
    nbt, ning, np_ = hopping_params.shape

    cols = []
    for tbl in (hopping_params, overlap_params):
        for k in range(np_):
            cols.append(jnp.pad(tbl[:, :, k], ((0, 0), (0, 16 - ning))))
    wt = jnp.concatenate(cols, axis=1).T
    rad = jnp.zeros((1, 128), f32).at[0, :atomic_radius_list.shape[0]].set(
        atomic_radius_list)
    eye = jnp.eye(blk, dtype=f32)

    r3 = r.reshape(nblk, 1, blk)
    bt3 = bond_indices.reshape(nblk, 1, blk)
    e03 = edge_numbers[0].reshape(nblk, 1, blk)
    e13 = edge_numbers[1].reshape(nblk, 1, blk)

    row_spec = pl.BlockSpec((1, 1, blk), lambda i: (i, 0, 0))
    out_spec = pl.BlockSpec((blk, ning), lambda i: (i, 0))
    hop, ov = pl.pallas_call(
        _body,
        grid=(nblk,),
        in_specs=[row_spec, row_spec, row_spec, row_spec,
                  pl.BlockSpec((128, nbt), lambda i: (0, 0)),
                  pl.BlockSpec((1, 128), lambda i: (0, 0)),
                  pl.BlockSpec((blk, blk), lambda i: (0, 0))],
        out_specs=[out_spec, out_spec],
        out_shape=[jax.ShapeDtypeStruct((B, ning), f32),
                   jax.ShapeDtypeStruct((B, ning), f32)],
    )(r3, bt3, e03, e13, wt, rad, eye)
    return (hop, ov)


# speedup vs baseline: 36.6173x; 2.1437x over previous
import jax
import jax.numpy as jnp
from jax import lax
from jax.experimental import pallas as pl

_BLK = 1024
_NING = 13


def _body(r_ref, bt_ref, e0_ref, e1_ref, wt_ref, rad_ref, hop_ref, ov_ref):
    f32 = jnp.float32
    rrow = r_ref[0]
    bt = bt_ref[0]
    e0 = e0_ref[0]
    e1 = e1_ref[0]
    nbt = wt_ref.shape[1]
    blk = rrow.shape[1]

    iota_t = lax.broadcasted_iota(jnp.int32, (nbt, blk), 0)
    oht = jnp.where(iota_t == bt, 1.0, 0.0).astype(f32)
    iota_e = lax.broadcasted_iota(jnp.int32, (128, blk), 0)
    ohe = jnp.where(iota_e == e0, 1.0, 0.0) + jnp.where(iota_e == e1, 1.0, 0.0)

    g = lax.dot_general(wt_ref[...], oht, (((1,), (0,)), ((), ())),
                        preferred_element_type=f32)
    r0 = lax.dot_general(rad_ref[...], ohe, (((1,), (0,)), ((), ())),
                         preferred_element_type=f32)

    x = rrow / r0 - 1.0
    x2 = x * x
    lnr = jnp.log(r0 / rrow)
    fcut = 1.0 / (1.0 + jnp.exp((rrow - 5.0) * 5.0))

    for t, out_ref in ((0, hop_ref), (1, ov_ref)):
        base = 64 * t
        g0 = g[base:base + 16]
        g1 = g[base + 16:base + 32]
        g2 = g[base + 32:base + 48]
        g3 = g[base + 48:base + 64]
        poly = g0 + g1 * x + g2 * x2
        a3 = 1.0 + jnp.abs(g3)
        out_t = poly * jnp.exp(a3 * lnr) * fcut
        out = out_t.T
        out_ref[...] = out[:, :_NING]


def kernel(r, bond_indices, edge_numbers, hopping_params, overlap_params,
           atomic_radius_list):
    f32 = jnp.float32
    B = r.shape[0]
    blk = _BLK
    nblk = B // blk
    nbt, ning, np_ = hopping_params.shape

    cols = []
    for tbl in (hopping_params, overlap_params):
        for k in range(np_):
            cols.append(jnp.pad(tbl[:, :, k], ((0, 0), (0, 16 - ning))))
    wt = jnp.concatenate(cols, axis=1).T
    rad = jnp.zeros((1, 128), f32).at[0, :atomic_radius_list.shape[0]].set(
        atomic_radius_list)

    r3 = r.reshape(nblk, 1, blk)
    bt3 = bond_indices.reshape(nblk, 1, blk)
    e03 = edge_numbers[0].reshape(nblk, 1, blk)
    e13 = edge_numbers[1].reshape(nblk, 1, blk)

    row_spec = pl.BlockSpec((1, 1, blk), lambda i: (i, 0, 0))
    out_spec = pl.BlockSpec((blk, ning), lambda i: (i, 0))
    hop, ov = pl.pallas_call(
        _body,
        grid=(nblk,),
        in_specs=[row_spec, row_spec, row_spec, row_spec,
                  pl.BlockSpec((128, nbt), lambda i: (0, 0)),
                  pl.BlockSpec((1, 128), lambda i: (0, 0))],
        out_specs=[out_spec, out_spec],
        out_shape=[jax.ShapeDtypeStruct((B, ning), f32),
                   jax.ShapeDtypeStruct((B, ning), f32)],
    )(r3, bt3, e03, e13, wt, rad)
    return (hop, ov)


# speedup vs baseline: 45.5613x; 1.2443x over previous
import jax
import jax.numpy as jnp
from jax import lax
from jax.experimental import pallas as pl

_BLK = 2048
_NING = 13


def _body(r_ref, bt_ref, e0_ref, e1_ref, wt_ref, rad_ref, hop_ref, ov_ref):
    f32 = jnp.float32
    rrow = r_ref[0]
    bt = bt_ref[0]
    e0 = e0_ref[0]
    e1 = e1_ref[0]
    nbt = wt_ref.shape[1]
    blk = rrow.shape[1]

    iota_t = lax.broadcasted_iota(jnp.int32, (nbt, blk), 0)
    oht = jnp.where(iota_t == bt, 1.0, 0.0).astype(f32)
    iota_e = lax.broadcasted_iota(jnp.int32, (128, blk), 0)
    ohe = jnp.where(iota_e == e0, 1.0, 0.0) + jnp.where(iota_e == e1, 1.0, 0.0)

    g = lax.dot_general(wt_ref[...], oht, (((1,), (0,)), ((), ())),
                        preferred_element_type=f32)
    r0 = lax.dot_general(rad_ref[...], ohe, (((1,), (0,)), ((), ())),
                         preferred_element_type=f32)

    x = rrow / r0 - 1.0
    x2 = x * x
    lnr = jnp.log(r0 / rrow)
    fcut = 1.0 / (1.0 + jnp.exp((rrow - 5.0) * 5.0))

    for t, out_ref in ((0, hop_ref), (1, ov_ref)):
        base = 64 * t
        g0 = g[base:base + 16]
        g1 = g[base + 16:base + 32]
        g2 = g[base + 32:base + 48]
        g3 = g[base + 48:base + 64]
        poly = g0 + g1 * x + g2 * x2
        a3 = 1.0 + jnp.abs(g3)
        out_t = poly * jnp.exp(a3 * lnr) * fcut
        out = out_t.T
        out_ref[...] = out[:, :_NING]


def kernel(r, bond_indices, edge_numbers, hopping_params, overlap_params,
           atomic_radius_list):
    f32 = jnp.float32
    B = r.shape[0]
    blk = _BLK
    nblk = B // blk
    nbt, ning, np_ = hopping_params.shape

    cols = []
    for tbl in (hopping_params, overlap_params):
        for k in range(np_):
            cols.append(jnp.pad(tbl[:, :, k], ((0, 0), (0, 16 - ning))))
    wt = jnp.concatenate(cols, axis=1).T
    rad = jnp.zeros((1, 128), f32).at[0, :atomic_radius_list.shape[0]].set(
        atomic_radius_list)

    r3 = r.reshape(nblk, 1, blk)
    bt3 = bond_indices.reshape(nblk, 1, blk)
    e03 = edge_numbers[0].reshape(nblk, 1, blk)
    e13 = edge_numbers[1].reshape(nblk, 1, blk)

    row_spec = pl.BlockSpec((1, 1, blk), lambda i: (i, 0, 0))
    out_spec = pl.BlockSpec((blk, ning), lambda i: (i, 0))
    hop, ov = pl.pallas_call(
        _body,
        grid=(nblk,),
        in_specs=[row_spec, row_spec, row_spec, row_spec,
                  pl.BlockSpec((128, nbt), lambda i: (0, 0)),
                  pl.BlockSpec((1, 128), lambda i: (0, 0))],
        out_specs=[out_spec, out_spec],
        out_shape=[jax.ShapeDtypeStruct((B, ning), f32),
                   jax.ShapeDtypeStruct((B, ning), f32)],
    )(r3, bt3, e03, e13, wt, rad)
    return (hop, ov)
